# baseline (speedup 1.0000x reference)
import functools

import jax
import jax.numpy as jnp
from jax import lax
from jax.experimental import pallas as pl
from jax.experimental.pallas import tpu as pltpu
from jax.experimental.pallas import tpu_sc as plsc

D = 64
B = 819200
NC = 2
NS = 16
NW = NC * NS
BPW = B // NW
SUB = 128
NSUB = BPW // SUB
GPC = 4
CHUNK = SUB * GPC
NCH = NSUB // GPC


def _sc_gather(idx3, table):
    mesh = plsc.VectorSubcoreMesh(core_axis_name="c", subcore_axis_name="s")

    @functools.partial(
        pl.kernel,
        mesh=mesh,
        compiler_params=pltpu.CompilerParams(use_tc_tiling_on_sc=False),
        out_type=jax.ShapeDtypeStruct((B, D), jnp.float32),
        scratch_types=[
            pltpu.VMEM((NSUB, SUB), jnp.int32),
            pltpu.VMEM((CHUNK, D), jnp.float32),
            pltpu.SemaphoreType.DMA,
        ],
    )
    def k(idx_hbm, table_hbm, out_hbm, idx_v, rows_v, sem):
        wid = lax.axis_index("s") * NC + lax.axis_index("c")
        pltpu.sync_copy(idx_hbm.at[wid], idx_v)
        base = wid * BPW

        def body(j, carry):
            cps = []
            for i in range(GPC):
                cps.append(pltpu.async_copy(
                    table_hbm.at[idx_v.at[j * GPC + i]],
                    rows_v.at[pl.ds(i * SUB, SUB)], sem))
            for cp in cps:
                cp.wait()
            pltpu.sync_copy(rows_v, out_hbm.at[pl.ds(base + j * CHUNK, CHUNK)])
            return carry

        lax.fori_loop(0, NCH, body, 0)

    return k(idx3, table)


def kernel(indices, batch_sizes, table):
    del batch_sizes
    idx3 = indices.astype(jnp.int32).reshape(NW, NSUB, SUB)
    return _sc_gather(idx3, table)


# speedup vs baseline: 1.0201x; 1.0201x over previous
import functools

import jax
import jax.numpy as jnp
from jax import lax
from jax.experimental import pallas as pl
from jax.experimental.pallas import tpu as pltpu
from jax.experimental.pallas import tpu_sc as plsc

D = 64
B = 819200
NC = 2
NS = 16
NW = NC * NS
BPW = B // NW
SUB = 128
GPC = 4
CHUNK = SUB * GPC
NSUB = BPW // SUB
NCH = NSUB // GPC


def _sc_gather(idx3, table):
    mesh = plsc.VectorSubcoreMesh(core_axis_name="c", subcore_axis_name="s")

    @functools.partial(
        pl.kernel,
        mesh=mesh,
        compiler_params=pltpu.CompilerParams(use_tc_tiling_on_sc=False),
        out_type=jax.ShapeDtypeStruct((B, D), jnp.float32),
        scratch_types=[
            pltpu.VMEM((NSUB, SUB), jnp.int32),
            pltpu.VMEM((CHUNK, D), jnp.float32),
            pltpu.VMEM((CHUNK, D), jnp.float32),
            pltpu.SemaphoreType.DMA,
            pltpu.SemaphoreType.DMA,
            pltpu.SemaphoreType.DMA,
            pltpu.SemaphoreType.DMA,
        ],
    )
    def k(idx_hbm, table_hbm, out_hbm, idx_v, buf0, buf1,
          gsem0, gsem1, wsem0, wsem1):
        wid = lax.axis_index("s") * NC + lax.axis_index("c")
        pltpu.sync_copy(idx_hbm.at[wid], idx_v)
        base = wid * BPW
        bufs = (buf0, buf1)
        gsems = (gsem0, gsem1)
        wsems = (wsem0, wsem1)

        def issue_gather(c, b):
            for i in range(GPC):
                pltpu.async_copy(
                    table_hbm.at[idx_v.at[c * GPC + i]],
                    bufs[b].at[pl.ds(i * SUB, SUB)], gsems[b])

        def wait_gather(c, b):
            for i in range(GPC):
                pltpu.make_async_copy(
                    table_hbm.at[idx_v.at[c * GPC + i]],
                    bufs[b].at[pl.ds(i * SUB, SUB)], gsems[b]).wait()

        def issue_write(c, b):
            pltpu.async_copy(
                bufs[b], out_hbm.at[pl.ds(base + c * CHUNK, CHUNK)], wsems[b])

        def wait_write(c, b):
            pltpu.make_async_copy(
                bufs[b], out_hbm.at[pl.ds(base + c * CHUNK, CHUNK)],
                wsems[b]).wait()

        issue_gather(0, 0)
        wait_gather(0, 0)
        issue_write(0, 0)
        issue_gather(1, 1)

        def body(p, carry):
            c = 1 + 2 * p
            wait_gather(c, 1)
            issue_write(c, 1)
            wait_write(c - 1, 0)
            issue_gather(c + 1, 0)
            wait_gather(c + 1, 0)
            issue_write(c + 1, 0)
            wait_write(c, 1)
            issue_gather(c + 2, 1)
            return carry

        lax.fori_loop(0, (NCH - 2) // 2, body, 0)
        wait_gather(NCH - 1, 1)
        issue_write(NCH - 1, 1)
        wait_write(NCH - 2, 0)
        wait_write(NCH - 1, 1)

    return k(idx3, table)


def kernel(indices, batch_sizes, table):
    del batch_sizes
    idx3 = indices.astype(jnp.int32).reshape(NW, NSUB, SUB)
    return _sc_gather(idx3, table)


# speedup vs baseline: 1.0209x; 1.0007x over previous
import functools

import jax
import jax.numpy as jnp
from jax import lax
from jax.experimental import pallas as pl
from jax.experimental.pallas import tpu as pltpu
from jax.experimental.pallas import tpu_sc as plsc

D = 64
B = 819200
NC = 2
NS = 16
NW = NC * NS
BPW = B // NW
SUB = 128
GPC = 4
CHUNK = SUB * GPC
NSUB = BPW // SUB
NCH = NSUB // GPC


def _sc_gather(idx3, table):
    mesh = plsc.VectorSubcoreMesh(core_axis_name="c", subcore_axis_name="s")

    @functools.partial(
        pl.kernel,
        mesh=mesh,
        compiler_params=pltpu.CompilerParams(use_tc_tiling_on_sc=False),
        out_type=jax.ShapeDtypeStruct((B // CHUNK, CHUNK, D), jnp.float32),
        scratch_types=[
            pltpu.VMEM((NSUB, SUB), jnp.int32),
            pltpu.VMEM((CHUNK, D), jnp.float32),
            pltpu.VMEM((CHUNK, D), jnp.float32),
            pltpu.SemaphoreType.DMA,
            pltpu.SemaphoreType.DMA,
            pltpu.SemaphoreType.DMA,
            pltpu.SemaphoreType.DMA,
        ],
    )
    def k(idx_hbm, table_hbm, out_hbm, idx_v, buf0, buf1,
          gsem0, gsem1, wsem0, wsem1):
        wid = lax.axis_index("s") * NC + lax.axis_index("c")
        pltpu.sync_copy(idx_hbm.at[wid], idx_v)
        base = wid * BPW
        bufs = (buf0, buf1)
        gsems = (gsem0, gsem1)
        wsems = (wsem0, wsem1)

        def issue_gather(c, b):
            for i in range(GPC):
                pltpu.async_copy(
                    table_hbm.at[idx_v.at[c * GPC + i]],
                    bufs[b].at[pl.ds(i * SUB, SUB)], gsems[b])

        def wait_gather(c, b):
            for i in range(GPC):
                pltpu.make_async_copy(
                    table_hbm.at[idx_v.at[c * GPC + i]],
                    bufs[b].at[pl.ds(i * SUB, SUB)], gsems[b]).wait()

        def issue_write(c, b):
            pltpu.async_copy(
                bufs[b],
                out_hbm.at[wid * NCH + c],
                wsems[b])

        def wait_write(c, b):
            pltpu.make_async_copy(
                bufs[b],
                out_hbm.at[wid * NCH + c],
                wsems[b]).wait()

        issue_gather(0, 0)
        wait_gather(0, 0)
        issue_write(0, 0)
        issue_gather(1, 1)

        def body(p, carry):
            c = 1 + 2 * p
            wait_gather(c, 1)
            issue_write(c, 1)
            wait_write(c - 1, 0)
            issue_gather(c + 1, 0)
            wait_gather(c + 1, 0)
            issue_write(c + 1, 0)
            wait_write(c, 1)
            issue_gather(c + 2, 1)
            return carry

        lax.fori_loop(0, (NCH - 2) // 2, body, 0)
        wait_gather(NCH - 1, 1)
        issue_write(NCH - 1, 1)
        wait_write(NCH - 2, 0)
        wait_write(NCH - 1, 1)

    return k(idx3, table)


def kernel(indices, batch_sizes, table):
    del batch_sizes
    idx3 = indices.astype(jnp.int32).reshape(NW, NSUB, SUB)
    return _sc_gather(idx3, table).reshape(B, D)
